# baseline (speedup 1.0000x reference)
import functools

import jax
import jax.numpy as jnp
from jax import lax
from jax.experimental import pallas as pl
from jax.experimental.pallas import tpu as pltpu
from jax.experimental.pallas import tpu_sc as plsc

BATCH = 32768
V = 32
NUM_BINS = 30
MIN_BIN_W = 0.001
MIN_BIN_H = 0.001
MIN_DERIV = 0.001
_EDGE_CONST = 0.5392745158594121

NC = 2
NS = 16
NW = NC * NS
ROWS_PER_TILE = BATCH // NW
ELEMS_PER_TILE = ROWS_PER_TILE * V

LUT = 1024
TBL = 1024
OFF_CB, OFF_CW, OFF_WI, OFF_CH, OFF_H, OFF_D = (k * TBL for k in range(6))

LN2 = 0.6931471805599453
_LOG2_C = (0.043928627847853945, -0.4094755857657619, 1.6101775468948172,
           -3.520218838142728, 5.069756316631331, -2.794153676535512)


def _tables_tc_kernel(uw_ref, uh_ref, ud_ref, tab_ref, lut_ref):
    uw = uw_ref[...]
    uh = uh_ref[...]
    ud = ud_ref[...]

    bi = lax.broadcasted_iota(jnp.int32, (NUM_BINS, V), 0)
    ji = lax.broadcasted_iota(jnp.int32, (NUM_BINS, V), 1)
    m = (bi < ji).astype(jnp.float32)

    def cumparams(u, min_size):
        mx = jnp.max(u, axis=-1, keepdims=True)
        e = jnp.exp(u - mx)
        p = e / jnp.sum(e, axis=-1, keepdims=True)
        p = min_size + (1.0 - min_size * NUM_BINS) * p
        cum = jnp.dot(p, m, preferred_element_type=jnp.float32,
                      precision=lax.Precision.HIGHEST)
        return cum

    cumw = cumparams(uw, MIN_BIN_W)
    cumh = cumparams(uh, MIN_BIN_H)
    w30 = cumw[:, 1:31] - cumw[:, 0:30]
    h30 = cumh[:, 1:31] - cumh[:, 0:30]
    ones2 = jnp.ones((V, 2), jnp.float32)

    col = lax.broadcasted_iota(jnp.int32, (V, V), 1)
    cb = jnp.where(col == 30, cumw + 1e-6, cumw)
    cb = jnp.where(col == 31, 1e30, cb)

    winv = jnp.concatenate([1.0 / w30, ones2], axis=1)
    hpad = jnp.concatenate([h30, ones2], axis=1)

    edge = jnp.full((V, 1), _EDGE_CONST, jnp.float32)
    dp31 = jnp.concatenate([edge, ud, edge], axis=1)
    sp = jnp.maximum(dp31, 0.0) + jnp.log(1.0 + jnp.exp(-jnp.abs(dp31)))
    dpad = jnp.concatenate([MIN_DERIV + sp, jnp.ones((V, 1), jnp.float32)],
                           axis=1)

    tab_ref[0] = cb
    tab_ref[1] = cumw
    tab_ref[2] = winv
    tab_ref[3] = cumh
    tab_ref[4] = hpad
    tab_ref[5] = dpad

    cells = (lax.broadcasted_iota(jnp.int32, (V, LUT), 1).astype(jnp.float32)
             * jnp.float32(1.0 / LUT))
    cnt = jnp.zeros((V, LUT), jnp.float32)
    for b in range(31):
        cnt = cnt + (cb[:, b:b + 1] <= cells).astype(jnp.float32)
    lut_ref[...] = jnp.clip(cnt.astype(jnp.int32) - 1, 0, NUM_BINS - 1)


def _build_tables(uw, uh, ud):
    return pl.pallas_call(
        _tables_tc_kernel,
        out_shape=(
            jax.ShapeDtypeStruct((6, V, V), jnp.float32),
            jax.ShapeDtypeStruct((V, LUT), jnp.int32),
        ),
    )(uw, uh, ud)


def _log2_poly(x):
    i = lax.bitcast_convert_type(x, jnp.int32)
    e = lax.convert_element_type(
        lax.shift_right_arithmetic(i, 23) - 127, jnp.float32)
    mb = lax.bitwise_or(lax.bitwise_and(i, 0x007FFFFF), 0x3F800000)
    mm = lax.bitcast_convert_type(mb, jnp.float32)
    p = jnp.full((16,), _LOG2_C[0], jnp.float32)
    for c in _LOG2_C[1:]:
        p = p * mm + c
    return e + p


SC_COLS = 16384
TC_COLS = BATCH - SC_COLS
TCB = 512

COLS_PER_TILE = SC_COLS // NW
NCHUNK = 4
CCOLS = COLS_PER_TILE // NCHUNK


def _tc_spline_kernel(x_ref, tab_ref, out_ref, det_ref):
    x = x_ref[...]
    tabs = tab_ref[...]
    cb = tabs[0]
    cwt = tabs[1]
    wit = tabs[2]
    cht = tabs[3]
    ht = tabs[4]
    dt = tabs[5]
    zero = jnp.zeros_like(x)
    one = jnp.ones_like(x)
    iprev = one
    cw = zero
    wi = zero
    ch = zero
    h = zero
    d = zero
    dp = zero
    for b in range(NUM_BINS):
        ib = jnp.where(x >= cb[:, b + 1:b + 2], 1.0, 0.0)
        oh = iprev - ib
        cw = cw + oh * cwt[:, b:b + 1]
        wi = wi + oh * wit[:, b:b + 1]
        ch = ch + oh * cht[:, b:b + 1]
        h = h + oh * ht[:, b:b + 1]
        d = d + oh * dt[:, b:b + 1]
        dp = dp + oh * dt[:, b + 1:b + 2]
        iprev = ib
    dl = h * wi
    th = (x - cw) * wi
    th2 = th * th
    th1 = th - th2
    num = h * (dl * th2 + d * th1)
    den = dl + (d + dp - 2.0 * dl) * th1
    rcp = 1.0 / den
    spl = ch + num * rcp
    omt = 1.0 - th
    nd = (dl * dl) * (dp * th2 + 2.0 * dl * th1 + d * (omt * omt))
    logdet = jnp.log(nd * (rcp * rcp))
    inside = jnp.logical_and(x >= 0.0, x <= 1.0)
    out_ref[...] = jnp.where(inside, spl, x)
    det_ref[...] = jnp.where(inside, logdet, zero)


def _tc_spline(xt, tab):
    return pl.pallas_call(
        _tc_spline_kernel,
        grid=(TC_COLS // TCB,),
        in_specs=[
            pl.BlockSpec((V, TCB), lambda i: (0, SC_COLS // TCB + i)),
            pl.BlockSpec((6, V, V), lambda i: (0, 0, 0)),
        ],
        out_specs=[
            pl.BlockSpec((V, TCB), lambda i: (0, i)),
            pl.BlockSpec((V, TCB), lambda i: (0, i)),
        ],
        out_shape=(
            jax.ShapeDtypeStruct((V, TC_COLS), jnp.float32),
            jax.ShapeDtypeStruct((V, TC_COLS), jnp.float32),
        ),
    )(xt, tab)


def _sc_body(x_hbm, tab_hbm, lut_hbm, out_hbm, det_hbm,
             tab_v, lut_v, xin0, xin1, o0, o1, d0, d1,
             si0, si1, so0, so1, sd0, sd1):
    wid = lax.axis_index("s") * NC + lax.axis_index("c")
    c0 = wid * COLS_PER_TILE
    xin = (xin0, xin1)
    outs = (o0, o1)
    dets = (d0, d1)
    isem = (si0, si1)
    osem = (so0, so1)
    dsem = (sd0, sd1)

    def start_in(k):
        return pltpu.async_copy(
            x_hbm.at[:, pl.ds(c0 + k * CCOLS, CCOLS)], xin[k % 2],
            isem[k % 2])

    def start_out(k):
        return (
            pltpu.async_copy(
                outs[k % 2], out_hbm.at[:, pl.ds(c0 + k * CCOLS, CCOLS)],
                osem[k % 2]),
            pltpu.async_copy(
                dets[k % 2], det_hbm.at[:, pl.ds(c0 + k * CCOLS, CCOLS)],
                dsem[k % 2]),
        )

    in_h = start_in(0)
    pltpu.sync_copy(tab_hbm, tab_v)
    pltpu.sync_copy(lut_hbm, lut_v)

    def do_vec(xv, ov, dv, r, col, tb, lb):
        x = xv[r, pl.ds(col, 16)]
        cell = jnp.clip(
            lax.convert_element_type(x * jnp.float32(LUT), jnp.int32),
            0, LUT - 1)
        l = plsc.load_gather(lut_v, [cell + lb])
        c1 = plsc.load_gather(tab_v, [l + (tb + OFF_CB + 1)])
        binr = jnp.minimum(jnp.where(c1 <= x, l + 1, l), NUM_BINS - 1)
        gi = binr + tb
        cw = plsc.load_gather(tab_v, [gi + OFF_CW])
        winv = plsc.load_gather(tab_v, [gi + OFF_WI])
        ch = plsc.load_gather(tab_v, [gi + OFF_CH])
        h = plsc.load_gather(tab_v, [gi + OFF_H])
        d = plsc.load_gather(tab_v, [gi + OFF_D])
        dp = plsc.load_gather(tab_v, [gi + (OFF_D + 1)])
        dl = h * winv

        th = (x - cw) * winv
        th2 = th * th
        th1 = th - th2
        num = h * (dl * th2 + d * th1)
        den = dl + (d + dp - 2.0 * dl) * th1
        rcp = 1.0 / den
        spl = ch + num * rcp
        omt = 1.0 - th
        nd = (dl * dl) * (dp * th2 + 2.0 * dl * th1 + d * (omt * omt))
        logdet = LN2 * _log2_poly(nd * (rcp * rcp))
        inside = jnp.logical_and(x >= 0.0, x <= 1.0)
        ov[r, pl.ds(col, 16)] = jnp.where(inside, spl, x)
        dv[r, pl.ds(col, 16)] = jnp.where(
            inside, logdet, jnp.zeros((16,), jnp.float32))

    vec_per_row = CCOLS // 16
    shift = vec_per_row.bit_length() - 1
    out_h = {}
    for k in range(NCHUNK):
        in_h.wait()
        if k + 1 < NCHUNK:
            in_h = start_in(k + 1)
        if k >= 2:
            for hh in out_h[k - 2]:
                hh.wait()
        xv, ov, dv = xin[k % 2], outs[k % 2], dets[k % 2]

        @plsc.parallel_loop(0, V * vec_per_row, unroll=4)
        def _loop(i, xv=xv, ov=ov, dv=dv):
            r = lax.shift_right_logical(i, shift)
            col = lax.bitwise_and(i, vec_per_row - 1) * 16
            do_vec(xv, ov, dv, r, col, r * V, r * LUT)

        out_h[k] = start_out(k)

    for k in (NCHUNK - 2, NCHUNK - 1):
        for hh in out_h[k]:
            hh.wait()


@functools.partial(jax.jit)
def _sc_spline(xt, tab_flat, lut_flat):
    mesh = plsc.VectorSubcoreMesh(core_axis_name="c", subcore_axis_name="s")
    f = functools.partial(
        pl.kernel,
        out_type=[
            jax.ShapeDtypeStruct((V, BATCH), jnp.float32),
            jax.ShapeDtypeStruct((V, BATCH), jnp.float32),
        ],
        mesh=mesh,
        compiler_params=pltpu.CompilerParams(needs_layout_passes=False),
        scratch_types=[
            pltpu.VMEM((6 * TBL,), jnp.float32),
            pltpu.VMEM((V * LUT,), jnp.int32),
            pltpu.VMEM((V, CCOLS), jnp.float32),
            pltpu.VMEM((V, CCOLS), jnp.float32),
            pltpu.VMEM((V, CCOLS), jnp.float32),
            pltpu.VMEM((V, CCOLS), jnp.float32),
            pltpu.VMEM((V, CCOLS), jnp.float32),
            pltpu.VMEM((V, CCOLS), jnp.float32),
            pltpu.SemaphoreType.DMA,
            pltpu.SemaphoreType.DMA,
            pltpu.SemaphoreType.DMA,
            pltpu.SemaphoreType.DMA,
            pltpu.SemaphoreType.DMA,
            pltpu.SemaphoreType.DMA,
        ],
    )(_sc_body)
    return f(xt, tab_flat, lut_flat)


def kernel(inputs, unnormalized_widths, unnormalized_heights,
           unnormalized_derivatives):
    tab, lut = _build_tables(unnormalized_widths, unnormalized_heights,
                             unnormalized_derivatives)
    xt = inputs.T
    sc_out, sc_det = _sc_spline(xt, tab.reshape(-1), lut.reshape(-1))
    tc_out, tc_det = _tc_spline(xt, tab)
    out_t = lax.dynamic_update_slice(sc_out, tc_out, (0, SC_COLS))
    det_t = lax.dynamic_update_slice(sc_det, tc_det, (0, SC_COLS))
    return (out_t.T, det_t.T)


# speedup vs baseline: 1.2234x; 1.2234x over previous
import functools

import jax
import jax.numpy as jnp
from jax import lax
from jax.experimental import pallas as pl
from jax.experimental.pallas import tpu as pltpu
from jax.experimental.pallas import tpu_sc as plsc

BATCH = 32768
V = 32
NUM_BINS = 30
MIN_BIN_W = 0.001
MIN_BIN_H = 0.001
MIN_DERIV = 0.001
_EDGE_CONST = 0.5392745158594121

NC = 2
NS = 16
NW = NC * NS
ROWS_PER_TILE = BATCH // NW
ELEMS_PER_TILE = ROWS_PER_TILE * V

LUT = 1024
TBL = 1024
OFF_CB, OFF_CW, OFF_WI, OFF_CH, OFF_H, OFF_D = (k * TBL for k in range(6))

LN2 = 0.6931471805599453
_LOG2_C = (0.043928627847853945, -0.4094755857657619, 1.6101775468948172,
           -3.520218838142728, 5.069756316631331, -2.794153676535512)


def _tables_tc_kernel(uw_ref, uh_ref, ud_ref, tab_ref, lut_ref):
    uw = uw_ref[...]
    uh = uh_ref[...]
    ud = ud_ref[...]

    bi = lax.broadcasted_iota(jnp.int32, (NUM_BINS, V), 0)
    ji = lax.broadcasted_iota(jnp.int32, (NUM_BINS, V), 1)
    m = (bi < ji).astype(jnp.float32)

    def cumparams(u, min_size):
        mx = jnp.max(u, axis=-1, keepdims=True)
        e = jnp.exp(u - mx)
        p = e / jnp.sum(e, axis=-1, keepdims=True)
        p = min_size + (1.0 - min_size * NUM_BINS) * p
        cum = jnp.dot(p, m, preferred_element_type=jnp.float32,
                      precision=lax.Precision.HIGHEST)
        return cum

    cumw = cumparams(uw, MIN_BIN_W)
    cumh = cumparams(uh, MIN_BIN_H)
    w30 = cumw[:, 1:31] - cumw[:, 0:30]
    h30 = cumh[:, 1:31] - cumh[:, 0:30]
    ones2 = jnp.ones((V, 2), jnp.float32)

    col = lax.broadcasted_iota(jnp.int32, (V, V), 1)
    cb = jnp.where(col == 30, cumw + 1e-6, cumw)
    cb = jnp.where(col == 31, 1e30, cb)

    winv = jnp.concatenate([1.0 / w30, ones2], axis=1)
    hpad = jnp.concatenate([h30, ones2], axis=1)

    edge = jnp.full((V, 1), _EDGE_CONST, jnp.float32)
    dp31 = jnp.concatenate([edge, ud, edge], axis=1)
    sp = jnp.maximum(dp31, 0.0) + jnp.log(1.0 + jnp.exp(-jnp.abs(dp31)))
    dpad = jnp.concatenate([MIN_DERIV + sp, jnp.ones((V, 1), jnp.float32)],
                           axis=1)

    tab_ref[0] = cb
    tab_ref[1] = cumw
    tab_ref[2] = winv
    tab_ref[3] = cumh
    tab_ref[4] = hpad
    tab_ref[5] = dpad

    cells = (lax.broadcasted_iota(jnp.int32, (V, LUT), 1).astype(jnp.float32)
             * jnp.float32(1.0 / LUT))
    cnt = jnp.zeros((V, LUT), jnp.float32)
    for b in range(31):
        cnt = cnt + (cb[:, b:b + 1] <= cells).astype(jnp.float32)
    lut_ref[...] = jnp.clip(cnt.astype(jnp.int32) - 1, 0, NUM_BINS - 1)


def _build_tables(uw, uh, ud):
    return pl.pallas_call(
        _tables_tc_kernel,
        out_shape=(
            jax.ShapeDtypeStruct((6, V, V), jnp.float32),
            jax.ShapeDtypeStruct((V, LUT), jnp.int32),
        ),
    )(uw, uh, ud)


def _log2_poly(x):
    i = lax.bitcast_convert_type(x, jnp.int32)
    e = lax.convert_element_type(
        lax.shift_right_arithmetic(i, 23) - 127, jnp.float32)
    mb = lax.bitwise_or(lax.bitwise_and(i, 0x007FFFFF), 0x3F800000)
    mm = lax.bitcast_convert_type(mb, jnp.float32)
    p = jnp.full((16,), _LOG2_C[0], jnp.float32)
    for c in _LOG2_C[1:]:
        p = p * mm + c
    return e + p


SC_COLS = 24576
TC_COLS = BATCH - SC_COLS
TCB = 512

COLS_PER_TILE = SC_COLS // NW
NCHUNK = 3
CCOLS = COLS_PER_TILE // NCHUNK


def _tc_spline_kernel(x_ref, tab_ref, out_ref, det_ref):
    x = x_ref[...]
    tabs = tab_ref[...]
    cb = tabs[0]
    cwt = tabs[1]
    wit = tabs[2]
    cht = tabs[3]
    ht = tabs[4]
    dt = tabs[5]
    zero = jnp.zeros_like(x)
    one = jnp.ones_like(x)
    iprev = one
    cw = zero
    wi = zero
    ch = zero
    h = zero
    d = zero
    dp = zero
    for b in range(NUM_BINS):
        ib = jnp.where(x >= cb[:, b + 1:b + 2], 1.0, 0.0)
        oh = iprev - ib
        cw = cw + oh * cwt[:, b:b + 1]
        wi = wi + oh * wit[:, b:b + 1]
        ch = ch + oh * cht[:, b:b + 1]
        h = h + oh * ht[:, b:b + 1]
        d = d + oh * dt[:, b:b + 1]
        dp = dp + oh * dt[:, b + 1:b + 2]
        iprev = ib
    dl = h * wi
    th = (x - cw) * wi
    th2 = th * th
    th1 = th - th2
    num = h * (dl * th2 + d * th1)
    den = dl + (d + dp - 2.0 * dl) * th1
    rcp = 1.0 / den
    spl = ch + num * rcp
    omt = 1.0 - th
    nd = (dl * dl) * (dp * th2 + 2.0 * dl * th1 + d * (omt * omt))
    logdet = jnp.log(nd * (rcp * rcp))
    inside = jnp.logical_and(x >= 0.0, x <= 1.0)
    out_ref[...] = jnp.where(inside, spl, x)
    det_ref[...] = jnp.where(inside, logdet, zero)


def _tc_spline(xt, tab):
    return pl.pallas_call(
        _tc_spline_kernel,
        grid=(TC_COLS // TCB,),
        in_specs=[
            pl.BlockSpec((V, TCB), lambda i: (0, SC_COLS // TCB + i)),
            pl.BlockSpec((6, V, V), lambda i: (0, 0, 0)),
        ],
        out_specs=[
            pl.BlockSpec((V, TCB), lambda i: (0, i)),
            pl.BlockSpec((V, TCB), lambda i: (0, i)),
        ],
        out_shape=(
            jax.ShapeDtypeStruct((V, TC_COLS), jnp.float32),
            jax.ShapeDtypeStruct((V, TC_COLS), jnp.float32),
        ),
    )(xt, tab)


def _sc_body(x_hbm, tab_hbm, lut_hbm, out_hbm, det_hbm,
             tab_v, lut_v, xin0, xin1, o0, o1, d0, d1,
             si0, si1, so0, so1, sd0, sd1):
    wid = lax.axis_index("s") * NC + lax.axis_index("c")
    c0 = wid * COLS_PER_TILE
    xin = (xin0, xin1)
    outs = (o0, o1)
    dets = (d0, d1)
    isem = (si0, si1)
    osem = (so0, so1)
    dsem = (sd0, sd1)

    def start_in(k):
        return pltpu.async_copy(
            x_hbm.at[:, pl.ds(c0 + k * CCOLS, CCOLS)], xin[k % 2],
            isem[k % 2])

    def start_out(k):
        return (
            pltpu.async_copy(
                outs[k % 2], out_hbm.at[:, pl.ds(c0 + k * CCOLS, CCOLS)],
                osem[k % 2]),
            pltpu.async_copy(
                dets[k % 2], det_hbm.at[:, pl.ds(c0 + k * CCOLS, CCOLS)],
                dsem[k % 2]),
        )

    in_h = start_in(0)
    pltpu.sync_copy(tab_hbm, tab_v)
    pltpu.sync_copy(lut_hbm, lut_v)

    def do_vec(xv, ov, dv, r, col, tb, lb):
        x = xv[r, pl.ds(col, 16)]
        cell = jnp.clip(
            lax.convert_element_type(x * jnp.float32(LUT), jnp.int32),
            0, LUT - 1)
        l = plsc.load_gather(lut_v, [cell + lb])
        c1 = plsc.load_gather(tab_v, [l + (tb + OFF_CB + 1)])
        binr = jnp.minimum(jnp.where(c1 <= x, l + 1, l), NUM_BINS - 1)
        gi = binr + tb
        cw = plsc.load_gather(tab_v, [gi + OFF_CW])
        winv = plsc.load_gather(tab_v, [gi + OFF_WI])
        ch = plsc.load_gather(tab_v, [gi + OFF_CH])
        h = plsc.load_gather(tab_v, [gi + OFF_H])
        d = plsc.load_gather(tab_v, [gi + OFF_D])
        dp = plsc.load_gather(tab_v, [gi + (OFF_D + 1)])
        dl = h * winv

        th = (x - cw) * winv
        th2 = th * th
        th1 = th - th2
        num = h * (dl * th2 + d * th1)
        den = dl + (d + dp - 2.0 * dl) * th1
        rcp = 1.0 / den
        spl = ch + num * rcp
        omt = 1.0 - th
        nd = (dl * dl) * (dp * th2 + 2.0 * dl * th1 + d * (omt * omt))
        logdet = LN2 * _log2_poly(nd * (rcp * rcp))
        inside = jnp.logical_and(x >= 0.0, x <= 1.0)
        ov[r, pl.ds(col, 16)] = jnp.where(inside, spl, x)
        dv[r, pl.ds(col, 16)] = jnp.where(
            inside, logdet, jnp.zeros((16,), jnp.float32))

    vec_per_row = CCOLS // 16
    shift = vec_per_row.bit_length() - 1
    out_h = {}
    for k in range(NCHUNK):
        in_h.wait()
        if k + 1 < NCHUNK:
            in_h = start_in(k + 1)
        if k >= 2:
            for hh in out_h[k - 2]:
                hh.wait()
        xv, ov, dv = xin[k % 2], outs[k % 2], dets[k % 2]

        @plsc.parallel_loop(0, V * vec_per_row, unroll=4)
        def _loop(i, xv=xv, ov=ov, dv=dv):
            r = lax.shift_right_logical(i, shift)
            col = lax.bitwise_and(i, vec_per_row - 1) * 16
            do_vec(xv, ov, dv, r, col, r * V, r * LUT)

        out_h[k] = start_out(k)

    for k in (NCHUNK - 2, NCHUNK - 1):
        for hh in out_h[k]:
            hh.wait()


@functools.partial(jax.jit)
def _sc_spline(xt, tab_flat, lut_flat):
    mesh = plsc.VectorSubcoreMesh(core_axis_name="c", subcore_axis_name="s")
    f = functools.partial(
        pl.kernel,
        out_type=[
            jax.ShapeDtypeStruct((V, BATCH), jnp.float32),
            jax.ShapeDtypeStruct((V, BATCH), jnp.float32),
        ],
        mesh=mesh,
        compiler_params=pltpu.CompilerParams(needs_layout_passes=False),
        scratch_types=[
            pltpu.VMEM((6 * TBL,), jnp.float32),
            pltpu.VMEM((V * LUT,), jnp.int32),
            pltpu.VMEM((V, CCOLS), jnp.float32),
            pltpu.VMEM((V, CCOLS), jnp.float32),
            pltpu.VMEM((V, CCOLS), jnp.float32),
            pltpu.VMEM((V, CCOLS), jnp.float32),
            pltpu.VMEM((V, CCOLS), jnp.float32),
            pltpu.VMEM((V, CCOLS), jnp.float32),
            pltpu.SemaphoreType.DMA,
            pltpu.SemaphoreType.DMA,
            pltpu.SemaphoreType.DMA,
            pltpu.SemaphoreType.DMA,
            pltpu.SemaphoreType.DMA,
            pltpu.SemaphoreType.DMA,
        ],
    )(_sc_body)
    return f(xt, tab_flat, lut_flat)


def kernel(inputs, unnormalized_widths, unnormalized_heights,
           unnormalized_derivatives):
    tab, lut = _build_tables(unnormalized_widths, unnormalized_heights,
                             unnormalized_derivatives)
    xt = inputs.T
    sc_out, sc_det = _sc_spline(xt, tab.reshape(-1), lut.reshape(-1))
    tc_out, tc_det = _tc_spline(xt, tab)
    out_t = lax.dynamic_update_slice(sc_out, tc_out, (0, SC_COLS))
    det_t = lax.dynamic_update_slice(sc_det, tc_det, (0, SC_COLS))
    return (out_t.T, det_t.T)
